# Initial kernel scaffold; baseline (speedup 1.0000x reference)
import jax
import jax.numpy as jnp
from jax.experimental import pallas as pl


def kernel(x, rbf, idx_i, idx_j, Wk2f, Wi, bi, Wj, bj, i0_W1, i0_b1, i0_W2, i0_b2, i1_W1, i1_b1, i1_W2, i1_b2, a0_W1, a0_b1, a0_W2, a0_b2, a1_W1, a1_b1, a1_W2, a1_b2, Wd, bd, u):
    raise NotImplementedError("write your pallas kernel here")


# speedup vs baseline: 3.1955x; 3.1955x over previous
import functools

import jax
import jax.numpy as jnp
from jax import lax
from jax.experimental import pallas as pl
from jax.experimental.pallas import tpu as pltpu
from jax.experimental.pallas import tpu_sc as plsc

N = 10000
E = 320000
K = 64
F = 128

_NC = 2
_NS = 16
_L = 16
_NW = _NC * _NS
_CHUNK = 128
_NCHUNKS = E // _CHUNK
_ROWS_PER_TILE = 624
_TAIL = N - _NS * _ROWS_PER_TILE


def _mm_bias_body(x_ref, w_ref, b_ref, o_ref):
    o_ref[...] = (
        jnp.dot(x_ref[...], w_ref[...], preferred_element_type=jnp.float32)
        + b_ref[...]
    )


def _mm_bias(x, w, b2d, blk):
    n = x.shape[0]
    return pl.pallas_call(
        _mm_bias_body,
        grid=(n // blk,),
        in_specs=[
            pl.BlockSpec((blk, x.shape[1]), lambda i: (i, 0)),
            pl.BlockSpec(w.shape, lambda i: (0, 0)),
            pl.BlockSpec((1, w.shape[1]), lambda i: (0, 0)),
        ],
        out_specs=pl.BlockSpec((blk, w.shape[1]), lambda i: (i, 0)),
        out_shape=jax.ShapeDtypeStruct((n, w.shape[1]), jnp.float32),
    )(x, w, b2d)


def _g_body(r_ref, w_ref, o_ref):
    o_ref[...] = jnp.dot(r_ref[...], w_ref[...], preferred_element_type=jnp.float32)


def _g_mm(rbf, wk2f, blk=4000):
    return pl.pallas_call(
        _g_body,
        grid=(E // blk,),
        in_specs=[
            pl.BlockSpec((blk, K), lambda i: (i, 0)),
            pl.BlockSpec((K, F), lambda i: (0, 0)),
        ],
        out_specs=pl.BlockSpec((blk, F), lambda i: (i, 0)),
        out_shape=jax.ShapeDtypeStruct((E, F), jnp.float32),
    )(rbf, wk2f)


def _epi_body(x_ref, xja_ref, xjb_ref, wi_ref, bi_ref,
              i0w1, i0b1, i0w2, i0b2, i1w1, i1b1, i1w2, i1b2,
              a0w1, a0b1, a0w2, a0b2, a1w1, a1b1, a1w2, a1b2,
              wd_ref, bd_ref, u_ref, o_ref):
    def mm(a, w):
        return jnp.dot(a, w[...], preferred_element_type=jnp.float32)

    def res(v, w1, b1, w2, b2):
        return v + mm(mm(v, w1) + b1[...], w2) + b2[...]

    x = x_ref[...]
    m = mm(x, wi_ref) + bi_ref[...] + xja_ref[...] + xjb_ref[...]
    m = res(m, i0w1, i0b1, i0w2, i0b2)
    m = res(m, i1w1, i1b1, i1w2, i1b2)
    out = u_ref[...] * x + mm(m, wd_ref) + bd_ref[...]
    out = res(out, a0w1, a0b1, a0w2, a0b2)
    out = res(out, a1w1, a1b1, a1w2, a1b2)
    o_ref[...] = out


def _epilogue(x, xj2, wi, bi2, rws, wd, bd2, u2, blk=2000):
    ng = N // blk
    row_spec = pl.BlockSpec((blk, F), lambda i: (i, 0))
    xjb_spec = pl.BlockSpec((blk, F), lambda i: (i + ng, 0))
    w_spec = pl.BlockSpec((F, F), lambda i: (0, 0))
    b_spec = pl.BlockSpec((1, F), lambda i: (0, 0))
    rw_specs = []
    for _ in range(4):
        rw_specs += [w_spec, b_spec, w_spec, b_spec]
    return pl.pallas_call(
        _epi_body,
        grid=(ng,),
        in_specs=[row_spec, row_spec, xjb_spec, w_spec, b_spec]
        + rw_specs + [w_spec, b_spec, b_spec],
        out_specs=row_spec,
        out_shape=jax.ShapeDtypeStruct((N, F), jnp.float32),
    )(x, xj2, xj2, wi, bi2, *rws, wd, bd2, u2)


def _sc_edge(g, xj_src, idx_i, idx_j, zrows):
    mesh = plsc.VectorSubcoreMesh(core_axis_name="c", subcore_axis_name="s")

    @functools.partial(
        pl.kernel,
        out_type=jax.ShapeDtypeStruct((2 * N, F), jnp.float32),
        mesh=mesh,
        scratch_types=[
            pltpu.VMEM((_CHUNK,), jnp.int32),
            pltpu.VMEM((_CHUNK,), jnp.int32),
            pltpu.VMEM((_CHUNK, F), jnp.float32),
            pltpu.VMEM((_CHUNK, F), jnp.float32),
            pltpu.VMEM_SHARED((N, F), jnp.float32),
            pltpu.SemaphoreType.DMA,
        ],
    )
    def k(g_hbm, xj_hbm, ii_hbm, ij_hbm, z_hbm, out_hbm,
          ii_v, ij_v, rows_v, g_v, acc_sh, sem):
        c = lax.axis_index("c")
        s = lax.axis_index("s")
        w = s * _NC + c

        pltpu.sync_copy(
            z_hbm.at[pl.ds(0, _ROWS_PER_TILE)],
            acc_sh.at[pl.ds(s * _ROWS_PER_TILE, _ROWS_PER_TILE)],
        )

        @pl.when(s == _NS - 1)
        def _zero_tail():
            pltpu.sync_copy(
                z_hbm.at[pl.ds(0, _TAIL)],
                acc_sh.at[pl.ds(_NS * _ROWS_PER_TILE, _TAIL)],
            )

        plsc.subcore_barrier()

        base_chunks = _NCHUNKS // _NW
        nch = jnp.where(w < _NCHUNKS % _NW, base_chunks + 1, base_chunks)

        @pl.loop(0, nch)
        def _chunk(t):
            base = (w + t * _NW) * _CHUNK
            pltpu.sync_copy(ij_hbm.at[pl.ds(base, _CHUNK)], ij_v)
            gat = pltpu.async_copy(xj_hbm.at[ij_v], rows_v, sem)
            pltpu.sync_copy(ii_hbm.at[pl.ds(base, _CHUNK)], ii_v)
            pltpu.sync_copy(g_hbm.at[pl.ds(base, _CHUNK)], g_v)
            gat.wait()

            @plsc.parallel_loop(0, _CHUNK * (F // _L), unroll=8)
            def _mul(t2):
                e = t2 // (F // _L)
                q = lax.rem(t2, F // _L) * _L
                g_v[e, pl.ds(q, _L)] = g_v[e, pl.ds(q, _L)] * rows_v[e, pl.ds(q, _L)]

            pltpu.sync_copy(g_v, acc_sh.at[ii_v], add=True)

        plsc.subcore_barrier()
        pltpu.sync_copy(
            acc_sh.at[pl.ds(s * _ROWS_PER_TILE, _ROWS_PER_TILE)],
            out_hbm.at[pl.ds(c * N + s * _ROWS_PER_TILE, _ROWS_PER_TILE)],
        )

        @pl.when(s == _NS - 1)
        def _out_tail():
            pltpu.sync_copy(
                acc_sh.at[pl.ds(_NS * _ROWS_PER_TILE, _TAIL)],
                out_hbm.at[pl.ds(c * N + _NS * _ROWS_PER_TILE, _TAIL)],
            )

    return k(g, xj_src, idx_i, idx_j, zrows)


def kernel(x, rbf, idx_i, idx_j, Wk2f, Wi, bi, Wj, bj,
           i0_W1, i0_b1, i0_W2, i0_b2, i1_W1, i1_b1, i1_W2, i1_b2,
           a0_W1, a0_b1, a0_W2, a0_b2, a1_W1, a1_b1, a1_W2, a1_b2,
           Wd, bd, u):
    xj_src = _mm_bias(x, Wj, bj.reshape(1, F), blk=2000)
    g = _g_mm(rbf, Wk2f)
    zrows = jnp.zeros((_ROWS_PER_TILE, F), jnp.float32)
    xj2 = _sc_edge(g, xj_src, idx_i.astype(jnp.int32), idx_j.astype(jnp.int32),
                   zrows)
    rws = (i0_W1, i0_b1.reshape(1, F), i0_W2, i0_b2.reshape(1, F),
           i1_W1, i1_b1.reshape(1, F), i1_W2, i1_b2.reshape(1, F),
           a0_W1, a0_b1.reshape(1, F), a0_W2, a0_b2.reshape(1, F),
           a1_W1, a1_b1.reshape(1, F), a1_W2, a1_b2.reshape(1, F))
    return _epilogue(x, xj2, Wi, bi.reshape(1, F), rws, Wd, bd.reshape(1, F),
                     u.reshape(1, F))


# speedup vs baseline: 3.2856x; 1.0282x over previous
import functools

import jax
import jax.numpy as jnp
from jax import lax
from jax.experimental import pallas as pl
from jax.experimental.pallas import tpu as pltpu
from jax.experimental.pallas import tpu_sc as plsc

N = 10000
E = 320000
K = 64
F = 128

_NC = 2
_NS = 16
_L = 16
_NW = _NC * _NS
_CHUNK = 80
_NCHUNKS = E // _CHUNK
_ROWS_PER_TILE = 624
_TAIL = N - _NS * _ROWS_PER_TILE


def _mm_bias_body(x_ref, w_ref, b_ref, o_ref):
    o_ref[...] = (
        jnp.dot(x_ref[...], w_ref[...], preferred_element_type=jnp.float32)
        + b_ref[...]
    )


def _mm_bias(x, w, b2d, blk):
    n = x.shape[0]
    return pl.pallas_call(
        _mm_bias_body,
        grid=(n // blk,),
        in_specs=[
            pl.BlockSpec((blk, x.shape[1]), lambda i: (i, 0)),
            pl.BlockSpec(w.shape, lambda i: (0, 0)),
            pl.BlockSpec((1, w.shape[1]), lambda i: (0, 0)),
        ],
        out_specs=pl.BlockSpec((blk, w.shape[1]), lambda i: (i, 0)),
        out_shape=jax.ShapeDtypeStruct((n, w.shape[1]), jnp.float32),
    )(x, w, b2d)


def _g_body(r_ref, w_ref, o_ref):
    o_ref[...] = jnp.dot(r_ref[...], w_ref[...], preferred_element_type=jnp.float32)


def _g_mm(rbf, wk2f, blk=4000):
    return pl.pallas_call(
        _g_body,
        grid=(E // blk,),
        in_specs=[
            pl.BlockSpec((blk, K), lambda i: (i, 0)),
            pl.BlockSpec((K, F), lambda i: (0, 0)),
        ],
        out_specs=pl.BlockSpec((blk, F), lambda i: (i, 0)),
        out_shape=jax.ShapeDtypeStruct((E, F), jnp.float32),
    )(rbf, wk2f)


def _epi_body(x_ref, xja_ref, xjb_ref, wi_ref, bi_ref,
              i0w1, i0b1, i0w2, i0b2, i1w1, i1b1, i1w2, i1b2,
              a0w1, a0b1, a0w2, a0b2, a1w1, a1b1, a1w2, a1b2,
              wd_ref, bd_ref, u_ref, o_ref):
    def mm(a, w):
        return jnp.dot(a, w[...], preferred_element_type=jnp.float32)

    def res(v, w1, b1, w2, b2):
        return v + mm(mm(v, w1) + b1[...], w2) + b2[...]

    x = x_ref[...]
    m = mm(x, wi_ref) + bi_ref[...] + xja_ref[...] + xjb_ref[...]
    m = res(m, i0w1, i0b1, i0w2, i0b2)
    m = res(m, i1w1, i1b1, i1w2, i1b2)
    out = u_ref[...] * x + mm(m, wd_ref) + bd_ref[...]
    out = res(out, a0w1, a0b1, a0w2, a0b2)
    out = res(out, a1w1, a1b1, a1w2, a1b2)
    o_ref[...] = out


def _epilogue(x, xj2, wi, bi2, rws, wd, bd2, u2, blk=2000):
    ng = N // blk
    row_spec = pl.BlockSpec((blk, F), lambda i: (i, 0))
    xjb_spec = pl.BlockSpec((blk, F), lambda i: (i + ng, 0))
    w_spec = pl.BlockSpec((F, F), lambda i: (0, 0))
    b_spec = pl.BlockSpec((1, F), lambda i: (0, 0))
    rw_specs = []
    for _ in range(4):
        rw_specs += [w_spec, b_spec, w_spec, b_spec]
    return pl.pallas_call(
        _epi_body,
        grid=(ng,),
        in_specs=[row_spec, row_spec, xjb_spec, w_spec, b_spec]
        + rw_specs + [w_spec, b_spec, b_spec],
        out_specs=row_spec,
        out_shape=jax.ShapeDtypeStruct((N, F), jnp.float32),
    )(x, xj2, xj2, wi, bi2, *rws, wd, bd2, u2)


def _sc_edge(g, xj_src, idx_i, idx_j, zrows):
    mesh = plsc.VectorSubcoreMesh(core_axis_name="c", subcore_axis_name="s")

    @functools.partial(
        pl.kernel,
        out_type=jax.ShapeDtypeStruct((2 * N, F), jnp.float32),
        mesh=mesh,
        scratch_types=[
            pltpu.VMEM((_CHUNK,), jnp.int32),
            pltpu.VMEM((_CHUNK,), jnp.int32),
            pltpu.VMEM((_CHUNK,), jnp.int32),
            pltpu.VMEM((_CHUNK,), jnp.int32),
            pltpu.VMEM((_CHUNK, F), jnp.float32),
            pltpu.VMEM((_CHUNK, F), jnp.float32),
            pltpu.VMEM((_CHUNK, F), jnp.float32),
            pltpu.VMEM((_CHUNK, F), jnp.float32),
            pltpu.VMEM_SHARED((N, F), jnp.float32),
            pltpu.SemaphoreType.DMA,
            pltpu.SemaphoreType.DMA,
            pltpu.SemaphoreType.DMA,
            pltpu.SemaphoreType.DMA,
        ],
    )
    def k(g_hbm, xj_hbm, ii_hbm, ij_hbm, z_hbm, out_hbm,
          ij0, ij1, ii0, ii1, rows0, rows1, g0, g1, acc_sh,
          sl0, sl1, sg0, sg1):
        c = lax.axis_index("c")
        s = lax.axis_index("s")
        w = s * _NC + c
        bufs = ((ij0, ii0, rows0, g0, sl0, sg0),
                (ij1, ii1, rows1, g1, sl1, sg1))

        pltpu.sync_copy(
            z_hbm.at[pl.ds(0, _ROWS_PER_TILE)],
            acc_sh.at[pl.ds(s * _ROWS_PER_TILE, _ROWS_PER_TILE)],
        )

        @pl.when(s == _NS - 1)
        def _zero_tail():
            pltpu.sync_copy(
                z_hbm.at[pl.ds(0, _TAIL)],
                acc_sh.at[pl.ds(_NS * _ROWS_PER_TILE, _TAIL)],
            )

        plsc.subcore_barrier()

        nch = _NCHUNKS // _NW

        def issue_linear(t, buf):
            ij_v, ii_v, _rows_v, g_v, sl, _sg = buf
            base = (w + t * _NW) * _CHUNK
            pltpu.async_copy(ij_hbm.at[pl.ds(base, _CHUNK)], ij_v, sl)
            pltpu.async_copy(ii_hbm.at[pl.ds(base, _CHUNK)], ii_v, sl)
            pltpu.async_copy(g_hbm.at[pl.ds(base, _CHUNK)], g_v, sl)

        def wait_linear_issue_gather(buf):
            ij_v, ii_v, rows_v, g_v, sl, sg = buf
            pltpu.make_async_copy(ij_hbm.at[pl.ds(0, _CHUNK)], ij_v, sl).wait()
            pltpu.make_async_copy(ii_hbm.at[pl.ds(0, _CHUNK)], ii_v, sl).wait()
            pltpu.make_async_copy(g_hbm.at[pl.ds(0, _CHUNK)], g_v, sl).wait()
            pltpu.async_copy(xj_hbm.at[ij_v], rows_v, sg)

        def finish(buf):
            ij_v, ii_v, rows_v, g_v, _sl, sg = buf
            pltpu.make_async_copy(xj_hbm.at[ij_v], rows_v, sg).wait()

            @plsc.parallel_loop(0, _CHUNK * (F // _L), unroll=8)
            def _mul(t2):
                e = t2 // (F // _L)
                q = lax.rem(t2, F // _L) * _L
                g_v[e, pl.ds(q, _L)] = g_v[e, pl.ds(q, _L)] * rows_v[e, pl.ds(q, _L)]

            pltpu.sync_copy(g_v, acc_sh.at[ii_v], add=True)

        issue_linear(jnp.int32(0), bufs[0])
        wait_linear_issue_gather(bufs[0])

        @pl.loop(0, nch // 2)
        def _pair(p):
            for b in range(2):
                t = 2 * p + b
                cur, nxt = bufs[b], bufs[1 - b]
                issue_linear(t + 1, nxt)
                finish(cur)
                wait_linear_issue_gather(nxt)

        finish(bufs[0])
        plsc.subcore_barrier()
        pltpu.sync_copy(
            acc_sh.at[pl.ds(s * _ROWS_PER_TILE, _ROWS_PER_TILE)],
            out_hbm.at[pl.ds(c * N + s * _ROWS_PER_TILE, _ROWS_PER_TILE)],
        )

        @pl.when(s == _NS - 1)
        def _out_tail():
            pltpu.sync_copy(
                acc_sh.at[pl.ds(_NS * _ROWS_PER_TILE, _TAIL)],
                out_hbm.at[pl.ds(c * N + _NS * _ROWS_PER_TILE, _TAIL)],
            )

    return k(g, xj_src, idx_i, idx_j, zrows)


def kernel(x, rbf, idx_i, idx_j, Wk2f, Wi, bi, Wj, bj,
           i0_W1, i0_b1, i0_W2, i0_b2, i1_W1, i1_b1, i1_W2, i1_b2,
           a0_W1, a0_b1, a0_W2, a0_b2, a1_W1, a1_b1, a1_W2, a1_b2,
           Wd, bd, u):
    xj_src = _mm_bias(x, Wj, bj.reshape(1, F), blk=2000)
    g = _g_mm(rbf, Wk2f)
    zrows = jnp.zeros((_ROWS_PER_TILE, F), jnp.float32)
    xj2 = _sc_edge(g, xj_src, idx_i.astype(jnp.int32), idx_j.astype(jnp.int32),
                   zrows)
    rws = (i0_W1, i0_b1.reshape(1, F), i0_W2, i0_b2.reshape(1, F),
           i1_W1, i1_b1.reshape(1, F), i1_W2, i1_b2.reshape(1, F),
           a0_W1, a0_b1.reshape(1, F), a0_W2, a0_b2.reshape(1, F),
           a1_W1, a1_b1.reshape(1, F), a1_W2, a1_b2.reshape(1, F))
    return _epilogue(x, xj2, Wi, bi.reshape(1, F), rws, Wd, bd.reshape(1, F),
                     u.reshape(1, F))


# speedup vs baseline: 3.5803x; 1.0897x over previous
import functools

import jax
import jax.numpy as jnp
from jax import lax
from jax.experimental import pallas as pl
from jax.experimental.pallas import tpu as pltpu
from jax.experimental.pallas import tpu_sc as plsc

N = 10000
E = 320000
K = 64
F = 128

_NC = 2
_NS = 16
_L = 16
_NW = _NC * _NS
_CHUNK = 80
_NCHUNKS = E // _CHUNK
_ROWS_PER_TILE = 624
_TAIL = N - _NS * _ROWS_PER_TILE


def _mm_bias_body(x_ref, w_ref, b_ref, o_ref):
    o_ref[...] = (
        jnp.dot(x_ref[...], w_ref[...], preferred_element_type=jnp.float32)
        + b_ref[...]
    )


def _mm_bias(x, w, b2d, blk):
    n = x.shape[0]
    return pl.pallas_call(
        _mm_bias_body,
        grid=(n // blk,),
        in_specs=[
            pl.BlockSpec((blk, x.shape[1]), lambda i: (i, 0)),
            pl.BlockSpec(w.shape, lambda i: (0, 0)),
            pl.BlockSpec((1, w.shape[1]), lambda i: (0, 0)),
        ],
        out_specs=pl.BlockSpec((blk, w.shape[1]), lambda i: (i, 0)),
        out_shape=jax.ShapeDtypeStruct((n, w.shape[1]), jnp.float32),
    )(x, w, b2d)


def _g_body(r_ref, w_ref, o_ref):
    o_ref[...] = jnp.dot(r_ref[...], w_ref[...], preferred_element_type=jnp.float32)


def _g_mm(rbf, wk2f, blk=4000):
    return pl.pallas_call(
        _g_body,
        grid=(E // blk,),
        in_specs=[
            pl.BlockSpec((blk, K), lambda i: (i, 0)),
            pl.BlockSpec((K, F), lambda i: (0, 0)),
        ],
        out_specs=pl.BlockSpec((blk, F), lambda i: (i, 0)),
        out_shape=jax.ShapeDtypeStruct((E, F), jnp.float32),
    )(rbf, wk2f)


def _epi_body(x_ref, xja_ref, xjb_ref, wi_ref, bi_ref,
              i0w1, i0b1, i0w2, i0b2, i1w1, i1b1, i1w2, i1b2,
              a0w1, a0b1, a0w2, a0b2, a1w1, a1b1, a1w2, a1b2,
              wd_ref, bd_ref, u_ref, o_ref):
    def mm(a, w):
        return jnp.dot(a, w[...], preferred_element_type=jnp.float32)

    def res(v, w1, b1, w2, b2):
        return v + mm(mm(v, w1) + b1[...], w2) + b2[...]

    x = x_ref[...]
    m = mm(x, wi_ref) + bi_ref[...] + xja_ref[...] + xjb_ref[...]
    m = res(m, i0w1, i0b1, i0w2, i0b2)
    m = res(m, i1w1, i1b1, i1w2, i1b2)
    out = u_ref[...] * x + mm(m, wd_ref) + bd_ref[...]
    out = res(out, a0w1, a0b1, a0w2, a0b2)
    out = res(out, a1w1, a1b1, a1w2, a1b2)
    o_ref[...] = out


def _epilogue(x, xj2, wi, bi2, rws, wd, bd2, u2, blk=2000):
    ng = N // blk
    row_spec = pl.BlockSpec((blk, F), lambda i: (i, 0))
    xjb_spec = pl.BlockSpec((blk, F), lambda i: (i + ng, 0))
    w_spec = pl.BlockSpec((F, F), lambda i: (0, 0))
    b_spec = pl.BlockSpec((1, F), lambda i: (0, 0))
    rw_specs = []
    for _ in range(4):
        rw_specs += [w_spec, b_spec, w_spec, b_spec]
    return pl.pallas_call(
        _epi_body,
        grid=(ng,),
        in_specs=[row_spec, row_spec, xjb_spec, w_spec, b_spec]
        + rw_specs + [w_spec, b_spec, b_spec],
        out_specs=row_spec,
        out_shape=jax.ShapeDtypeStruct((N, F), jnp.float32),
    )(x, xj2, xj2, wi, bi2, *rws, wd, bd2, u2)


def _sc_edge(g, xj_src, idx_i, idx_j, zrows):
    mesh = plsc.VectorSubcoreMesh(core_axis_name="c", subcore_axis_name="s")

    @functools.partial(
        pl.kernel,
        out_type=jax.ShapeDtypeStruct((2 * N, F), jnp.float32),
        mesh=mesh,
        scratch_types=[
            pltpu.VMEM((_CHUNK,), jnp.int32),
            pltpu.VMEM((_CHUNK,), jnp.int32),
            pltpu.VMEM((_CHUNK,), jnp.int32),
            pltpu.VMEM((_CHUNK,), jnp.int32),
            pltpu.VMEM((_CHUNK, F), jnp.float32),
            pltpu.VMEM((_CHUNK, F), jnp.float32),
            pltpu.VMEM((_CHUNK, F), jnp.float32),
            pltpu.VMEM((_CHUNK, F), jnp.float32),
            pltpu.VMEM_SHARED((N, F), jnp.float32),
            pltpu.SemaphoreType.DMA,
            pltpu.SemaphoreType.DMA,
            pltpu.SemaphoreType.DMA,
            pltpu.SemaphoreType.DMA,
            pltpu.SemaphoreType.DMA,
            pltpu.SemaphoreType.DMA,
        ],
    )
    def k(g_hbm, xj_hbm, ii_hbm, ij_hbm, z_hbm, out_hbm,
          ij0, ij1, ii0, ii1, rows0, rows1, g0, g1, acc_sh,
          sl0, sl1, sg0, sg1, ss0, ss1):
        c = lax.axis_index("c")
        s = lax.axis_index("s")
        w = s * _NC + c
        bufs = ((ij0, ii0, rows0, g0, sl0, sg0, ss0),
                (ij1, ii1, rows1, g1, sl1, sg1, ss1))

        pltpu.sync_copy(
            z_hbm.at[pl.ds(0, _ROWS_PER_TILE)],
            acc_sh.at[pl.ds(s * _ROWS_PER_TILE, _ROWS_PER_TILE)],
        )

        @pl.when(s == _NS - 1)
        def _zero_tail():
            pltpu.sync_copy(
                z_hbm.at[pl.ds(0, _TAIL)],
                acc_sh.at[pl.ds(_NS * _ROWS_PER_TILE, _TAIL)],
            )

        plsc.subcore_barrier()

        nch = _NCHUNKS // _NW

        def issue_linear(t, buf):
            ij_v, ii_v, _rows_v, g_v, sl, _sg, _ss = buf
            base = (w + t * _NW) * _CHUNK
            pltpu.async_copy(ij_hbm.at[pl.ds(base, _CHUNK)], ij_v, sl)
            pltpu.async_copy(ii_hbm.at[pl.ds(base, _CHUNK)], ii_v, sl)
            pltpu.async_copy(g_hbm.at[pl.ds(base, _CHUNK)], g_v, sl)

        def wait_linear_issue_gather(buf):
            ij_v, ii_v, rows_v, g_v, sl, sg, _ss = buf
            pltpu.make_async_copy(ij_hbm.at[pl.ds(0, _CHUNK)], ij_v, sl).wait()
            pltpu.make_async_copy(ii_hbm.at[pl.ds(0, _CHUNK)], ii_v, sl).wait()
            pltpu.make_async_copy(g_hbm.at[pl.ds(0, _CHUNK)], g_v, sl).wait()
            pltpu.async_copy(xj_hbm.at[ij_v], rows_v, sg)

        def wait_scatter(buf):
            ij_v, ii_v, rows_v, g_v, _sl, _sg, ss = buf
            pltpu.make_async_copy(g_v, acc_sh.at[ii_v], ss).wait()

        def finish(t, buf):
            ij_v, ii_v, rows_v, g_v, _sl, sg, ss = buf
            pltpu.make_async_copy(xj_hbm.at[ij_v], rows_v, sg).wait()

            @plsc.parallel_loop(0, _CHUNK * (F // _L), unroll=8)
            def _mul(t2):
                e = t2 // (F // _L)
                q = lax.rem(t2, F // _L) * _L
                g_v[e, pl.ds(q, _L)] = g_v[e, pl.ds(q, _L)] * rows_v[e, pl.ds(q, _L)]

            pltpu.async_copy(g_v, acc_sh.at[ii_v], ss, add=True)

        issue_linear(jnp.int32(0), bufs[0])
        wait_linear_issue_gather(bufs[0])

        @pl.loop(0, nch // 2)
        def _pair(p):
            for b in range(2):
                t = 2 * p + b
                cur, nxt = bufs[b], bufs[1 - b]
                issue_linear(t + 1, nxt)

                @pl.when(t >= 2)
                def _drain(cur=cur):
                    wait_scatter(cur)

                finish(t, cur)
                wait_linear_issue_gather(nxt)

        wait_scatter(bufs[0])
        finish(124, bufs[0])
        wait_scatter(bufs[1])
        wait_scatter(bufs[0])
        plsc.subcore_barrier()
        pltpu.sync_copy(
            acc_sh.at[pl.ds(s * _ROWS_PER_TILE, _ROWS_PER_TILE)],
            out_hbm.at[pl.ds(c * N + s * _ROWS_PER_TILE, _ROWS_PER_TILE)],
        )

        @pl.when(s == _NS - 1)
        def _out_tail():
            pltpu.sync_copy(
                acc_sh.at[pl.ds(_NS * _ROWS_PER_TILE, _TAIL)],
                out_hbm.at[pl.ds(c * N + _NS * _ROWS_PER_TILE, _TAIL)],
            )

    return k(g, xj_src, idx_i, idx_j, zrows)


def kernel(x, rbf, idx_i, idx_j, Wk2f, Wi, bi, Wj, bj,
           i0_W1, i0_b1, i0_W2, i0_b2, i1_W1, i1_b1, i1_W2, i1_b2,
           a0_W1, a0_b1, a0_W2, a0_b2, a1_W1, a1_b1, a1_W2, a1_b2,
           Wd, bd, u):
    xj_src = _mm_bias(x, Wj, bj.reshape(1, F), blk=2000)
    g = _g_mm(rbf, Wk2f)
    zrows = jnp.zeros((_ROWS_PER_TILE, F), jnp.float32)
    xj2 = _sc_edge(g, xj_src, idx_i.astype(jnp.int32), idx_j.astype(jnp.int32),
                   zrows)
    rws = (i0_W1, i0_b1.reshape(1, F), i0_W2, i0_b2.reshape(1, F),
           i1_W1, i1_b1.reshape(1, F), i1_W2, i1_b2.reshape(1, F),
           a0_W1, a0_b1.reshape(1, F), a0_W2, a0_b2.reshape(1, F),
           a1_W1, a1_b1.reshape(1, F), a1_W2, a1_b2.reshape(1, F))
    return _epilogue(x, xj2, Wi, bi.reshape(1, F), rws, Wd, bd.reshape(1, F),
                     u.reshape(1, F))


# speedup vs baseline: 3.8270x; 1.0689x over previous
import functools

import jax
import jax.numpy as jnp
from jax import lax
from jax.experimental import pallas as pl
from jax.experimental.pallas import tpu as pltpu
from jax.experimental.pallas import tpu_sc as plsc

N = 10000
E = 320000
K = 64
F = 128

_NC = 2
_NS = 16
_L = 16
_NW = _NC * _NS
_CHUNK = 80
_NCHUNKS = E // _CHUNK
_ROWS_PER_TILE = 624
_TAIL = N - _NS * _ROWS_PER_TILE


def _mm_bias_body(x_ref, w_ref, b_ref, o_ref):
    o_ref[...] = (
        jnp.dot(x_ref[...], w_ref[...], preferred_element_type=jnp.float32)
        + b_ref[...]
    )


def _mm_bias(x, w, b2d, blk):
    n = x.shape[0]
    return pl.pallas_call(
        _mm_bias_body,
        grid=(n // blk,),
        in_specs=[
            pl.BlockSpec((blk, x.shape[1]), lambda i: (i, 0)),
            pl.BlockSpec(w.shape, lambda i: (0, 0)),
            pl.BlockSpec((1, w.shape[1]), lambda i: (0, 0)),
        ],
        out_specs=pl.BlockSpec((blk, w.shape[1]), lambda i: (i, 0)),
        out_shape=jax.ShapeDtypeStruct((n, w.shape[1]), jnp.float32),
    )(x, w, b2d)


def _g_body(r_ref, w_ref, o_ref):
    o_ref[...] = jnp.dot(r_ref[...], w_ref[...], preferred_element_type=jnp.float32)


def _g_mm(rbf, wk2f, blk=4000):
    return pl.pallas_call(
        _g_body,
        grid=(E // blk,),
        in_specs=[
            pl.BlockSpec((blk, K), lambda i: (i, 0)),
            pl.BlockSpec((K, F), lambda i: (0, 0)),
        ],
        out_specs=pl.BlockSpec((blk, F), lambda i: (i, 0)),
        out_shape=jax.ShapeDtypeStruct((E, F), jnp.float32),
    )(rbf, wk2f)


def _epi_body(x_ref, xja_ref, xjb_ref, wi_ref, bi_ref,
              i0w1, i0b1, i0w2, i0b2, i1w1, i1b1, i1w2, i1b2,
              a0w1, a0b1, a0w2, a0b2, a1w1, a1b1, a1w2, a1b2,
              wd_ref, bd_ref, u_ref, o_ref):
    def mm(a, w):
        return jnp.dot(a, w[...], preferred_element_type=jnp.float32)

    def res(v, w1, b1, w2, b2):
        return v + mm(mm(v, w1) + b1[...], w2) + b2[...]

    x = x_ref[...]
    m = mm(x, wi_ref) + bi_ref[...] + xja_ref[...] + xjb_ref[...]
    m = res(m, i0w1, i0b1, i0w2, i0b2)
    m = res(m, i1w1, i1b1, i1w2, i1b2)
    out = u_ref[...] * x + mm(m, wd_ref) + bd_ref[...]
    out = res(out, a0w1, a0b1, a0w2, a0b2)
    out = res(out, a1w1, a1b1, a1w2, a1b2)
    o_ref[...] = out


def _epilogue(x, xj2, wi, bi2, rws, wd, bd2, u2, blk=2000):
    ng = N // blk
    row_spec = pl.BlockSpec((blk, F), lambda i: (i, 0))
    xjb_spec = pl.BlockSpec((blk, F), lambda i: (i + ng, 0))
    w_spec = pl.BlockSpec((F, F), lambda i: (0, 0))
    b_spec = pl.BlockSpec((1, F), lambda i: (0, 0))
    rw_specs = []
    for _ in range(4):
        rw_specs += [w_spec, b_spec, w_spec, b_spec]
    return pl.pallas_call(
        _epi_body,
        grid=(ng,),
        in_specs=[row_spec, row_spec, xjb_spec, w_spec, b_spec]
        + rw_specs + [w_spec, b_spec, b_spec],
        out_specs=row_spec,
        out_shape=jax.ShapeDtypeStruct((N, F), jnp.float32),
    )(x, xj2, xj2, wi, bi2, *rws, wd, bd2, u2)


def _sc_edge(g, xj_src, idx_i, idx_j, zrows):
    mesh = plsc.VectorSubcoreMesh(core_axis_name="c", subcore_axis_name="s")

    @functools.partial(
        pl.kernel,
        out_type=jax.ShapeDtypeStruct((2 * N, F), jnp.float32),
        mesh=mesh,
        scratch_types=[
            pltpu.VMEM((_CHUNK,), jnp.int32),
            pltpu.VMEM((_CHUNK,), jnp.int32),
            pltpu.VMEM((_CHUNK,), jnp.int32),
            pltpu.VMEM((_CHUNK,), jnp.int32),
            pltpu.VMEM((_CHUNK, F), jnp.float32),
            pltpu.VMEM((_CHUNK, F), jnp.float32),
            pltpu.VMEM((_CHUNK, F), jnp.float32),
            pltpu.VMEM((_CHUNK, F), jnp.float32),
            pltpu.VMEM_SHARED((N, F), jnp.float32),
            pltpu.SemaphoreType.DMA,
            pltpu.SemaphoreType.DMA,
            pltpu.SemaphoreType.DMA,
            pltpu.SemaphoreType.DMA,
            pltpu.SemaphoreType.DMA,
            pltpu.SemaphoreType.DMA,
        ],
    )
    def k(g_hbm, xj_hbm, ii_hbm, ij_hbm, z_hbm, out_hbm,
          ij0, ij1, ii0, ii1, rows0, rows1, g0, g1, acc_sh,
          sl0, sl1, sg0, sg1, ss0, ss1):
        c = lax.axis_index("c")
        s = lax.axis_index("s")
        w = s * _NC + c
        bufs = ((ij0, ii0, rows0, g0, sl0, sg0, ss0),
                (ij1, ii1, rows1, g1, sl1, sg1, ss1))

        pltpu.sync_copy(
            z_hbm.at[pl.ds(0, _ROWS_PER_TILE)],
            acc_sh.at[pl.ds(s * _ROWS_PER_TILE, _ROWS_PER_TILE)],
        )

        @pl.when(s == _NS - 1)
        def _zero_tail():
            pltpu.sync_copy(
                z_hbm.at[pl.ds(0, _TAIL)],
                acc_sh.at[pl.ds(_NS * _ROWS_PER_TILE, _TAIL)],
            )

        plsc.subcore_barrier()

        nch = _NCHUNKS // _NW

        def issue_linear(t, buf):
            ij_v, ii_v, _rows_v, g_v, sl, _sg, _ss = buf
            base = (w + t * _NW) * _CHUNK
            pltpu.async_copy(ij_hbm.at[pl.ds(base, _CHUNK)], ij_v, sl)
            pltpu.async_copy(ii_hbm.at[pl.ds(base, _CHUNK)], ii_v, sl)
            pltpu.async_copy(g_hbm.at[pl.ds(base, _CHUNK)], g_v, sl)

        def wait_linear_issue_gather(buf):
            ij_v, ii_v, rows_v, g_v, sl, sg, _ss = buf
            pltpu.make_async_copy(ij_hbm.at[pl.ds(0, _CHUNK)], ij_v, sl).wait()
            pltpu.make_async_copy(ii_hbm.at[pl.ds(0, _CHUNK)], ii_v, sl).wait()
            pltpu.make_async_copy(g_hbm.at[pl.ds(0, _CHUNK)], g_v, sl).wait()
            pltpu.async_copy(xj_hbm.at[ij_v], rows_v, sg)

        def wait_scatter(buf):
            ij_v, ii_v, rows_v, g_v, _sl, _sg, ss = buf
            pltpu.make_async_copy(g_v, acc_sh.at[ii_v], ss).wait()

        def finish(t, buf):
            ij_v, ii_v, rows_v, g_v, _sl, sg, ss = buf
            pltpu.make_async_copy(xj_hbm.at[ij_v], rows_v, sg).wait()

            @plsc.parallel_loop(0, _CHUNK * (F // _L), unroll=8)
            def _mul(t2):
                e = t2 // (F // _L)
                q = lax.rem(t2, F // _L) * _L
                g_v[e, pl.ds(q, _L)] = g_v[e, pl.ds(q, _L)] * rows_v[e, pl.ds(q, _L)]

            pltpu.async_copy(g_v, acc_sh.at[ii_v], ss, add=True)

        issue_linear(jnp.int32(0), bufs[0])
        wait_linear_issue_gather(bufs[0])

        @pl.loop(0, nch // 2)
        def _pair(p):
            for b in range(2):
                t = 2 * p + b
                cur, nxt = bufs[b], bufs[1 - b]

                if b == 0:
                    @pl.when(t >= 1)
                    def _drain(nxt=nxt):
                        wait_scatter(nxt)
                else:
                    wait_scatter(nxt)

                issue_linear(t + 1, nxt)
                finish(t, cur)
                wait_linear_issue_gather(nxt)

        wait_scatter(bufs[1])
        finish(124, bufs[0])
        wait_scatter(bufs[0])
        plsc.subcore_barrier()
        pltpu.sync_copy(
            acc_sh.at[pl.ds(s * _ROWS_PER_TILE, _ROWS_PER_TILE)],
            out_hbm.at[pl.ds(c * N + s * _ROWS_PER_TILE, _ROWS_PER_TILE)],
        )

        @pl.when(s == _NS - 1)
        def _out_tail():
            pltpu.sync_copy(
                acc_sh.at[pl.ds(_NS * _ROWS_PER_TILE, _TAIL)],
                out_hbm.at[pl.ds(c * N + _NS * _ROWS_PER_TILE, _TAIL)],
            )

    return k(g, xj_src, idx_i, idx_j, zrows)


def kernel(x, rbf, idx_i, idx_j, Wk2f, Wi, bi, Wj, bj,
           i0_W1, i0_b1, i0_W2, i0_b2, i1_W1, i1_b1, i1_W2, i1_b2,
           a0_W1, a0_b1, a0_W2, a0_b2, a1_W1, a1_b1, a1_W2, a1_b2,
           Wd, bd, u):
    xj_src = _mm_bias(x, Wj, bj.reshape(1, F), blk=2000)
    g = _g_mm(rbf, Wk2f)
    zrows = jnp.zeros((_ROWS_PER_TILE, F), jnp.float32)
    xj2 = _sc_edge(g, xj_src, idx_i.astype(jnp.int32), idx_j.astype(jnp.int32),
                   zrows)
    rws = (i0_W1, i0_b1.reshape(1, F), i0_W2, i0_b2.reshape(1, F),
           i1_W1, i1_b1.reshape(1, F), i1_W2, i1_b2.reshape(1, F),
           a0_W1, a0_b1.reshape(1, F), a0_W2, a0_b2.reshape(1, F),
           a1_W1, a1_b1.reshape(1, F), a1_W2, a1_b2.reshape(1, F))
    return _epilogue(x, xj2, Wi, bi.reshape(1, F), rws, Wd, bd.reshape(1, F),
                     u.reshape(1, F))
